# Initial kernel scaffold; baseline (speedup 1.0000x reference)
import jax
import jax.numpy as jnp
from jax.experimental import pallas as pl


def kernel(user, song, user_table, song_table, W1, b1, W2, b2, W3, b3):
    raise NotImplementedError("write your pallas kernel here")


# speedup vs baseline: 1.1534x; 1.1534x over previous
import functools

import jax
import jax.numpy as jnp
from jax import lax
from jax.experimental import pallas as pl
from jax.experimental.pallas import tpu as pltpu
from jax.experimental.pallas import tpu_sc as plsc

BATCH = 16384
EMBED_DIM = 64
H1 = 128
H2 = 64


@functools.cache
def _build_gather():
    info = plsc.get_sparse_core_info()
    nc, ns = info.num_cores, info.num_subcores
    nw = nc * ns
    bpw = BATCH // nw

    mesh = plsc.VectorSubcoreMesh(core_axis_name="c", subcore_axis_name="s")

    @functools.partial(
        pl.kernel,
        mesh=mesh,
        compiler_params=pltpu.CompilerParams(use_tc_tiling_on_sc=False),
        out_type=[
            jax.ShapeDtypeStruct((BATCH, EMBED_DIM), jnp.float32),
            jax.ShapeDtypeStruct((BATCH, EMBED_DIM), jnp.float32),
        ],
        scratch_types=[
            pltpu.VMEM((bpw,), jnp.int32),
            pltpu.VMEM((bpw, EMBED_DIM), jnp.float32),
            pltpu.VMEM((bpw,), jnp.int32),
            pltpu.VMEM((bpw, EMBED_DIM), jnp.float32),
            pltpu.SemaphoreType.DMA,
            pltpu.SemaphoreType.DMA,
        ],
    )
    def gather2(u_tab, s_tab, u_idx, s_idx, u_out, s_out,
                u_idx_v, u_rows_v, s_idx_v, s_rows_v, u_sem, s_sem):
        wid = lax.axis_index("s") * nc + lax.axis_index("c")
        base = wid * bpw
        pltpu.sync_copy(u_idx.at[pl.ds(base, bpw)], u_idx_v)
        pltpu.sync_copy(s_idx.at[pl.ds(base, bpw)], s_idx_v)
        u_cp = pltpu.async_copy(u_tab.at[u_idx_v], u_rows_v, u_sem)
        s_cp = pltpu.async_copy(s_tab.at[s_idx_v], s_rows_v, s_sem)
        u_cp.wait()
        pltpu.sync_copy(u_rows_v, u_out.at[pl.ds(base, bpw)])
        s_cp.wait()
        pltpu.sync_copy(s_rows_v, s_out.at[pl.ds(base, bpw)])

    return gather2


def _mlp_body(ue_ref, se_ref, w1u_ref, w1s_ref, b1_ref, w2_ref, b2_ref,
              w3_ref, b3_ref, out_ref):
    h = jnp.dot(ue_ref[...], w1u_ref[...], preferred_element_type=jnp.float32)
    h += jnp.dot(se_ref[...], w1s_ref[...], preferred_element_type=jnp.float32)
    h = jnp.maximum(h + b1_ref[...], 0.0)
    h = jnp.dot(h, w2_ref[...], preferred_element_type=jnp.float32)
    h = jnp.maximum(h + b2_ref[...], 0.0)
    o = jnp.dot(h, w3_ref[...], preferred_element_type=jnp.float32)
    o = o + b3_ref[...]
    out_ref[...] = 1.0 / (1.0 + jnp.exp(-o))


def _mlp(u_emb, s_emb, w1uT, w1sT, b1, w2T, b2, w3T, b3):
    blk = 2048
    grid = BATCH // blk
    full = lambda shape: pl.BlockSpec(shape, lambda i: (0, 0))
    return pl.pallas_call(
        _mlp_body,
        grid=(grid,),
        in_specs=[
            pl.BlockSpec((blk, EMBED_DIM), lambda i: (i, 0)),
            pl.BlockSpec((blk, EMBED_DIM), lambda i: (i, 0)),
            full((EMBED_DIM, H1)),
            full((EMBED_DIM, H1)),
            full((1, H1)),
            full((H1, H2)),
            full((1, H2)),
            full((H2, 1)),
            full((1, 1)),
        ],
        out_specs=pl.BlockSpec((blk, 1), lambda i: (i, 0)),
        out_shape=jax.ShapeDtypeStruct((BATCH, 1), jnp.float32),
    )(u_emb, s_emb, w1uT, w1sT, b1, w2T, b2, w3T, b3)


def kernel(user, song, user_table, song_table, W1, b1, W2, b2, W3, b3):
    u_emb, s_emb = _build_gather()(
        user_table, song_table,
        user.astype(jnp.int32), song.astype(jnp.int32))
    w1uT = W1[:, :EMBED_DIM].T
    w1sT = W1[:, EMBED_DIM:].T
    return _mlp(u_emb, s_emb, w1uT, w1sT, b1.reshape(1, H1),
                W2.T, b2.reshape(1, H2), W3.T, b3.reshape(1, 1))


# speedup vs baseline: 1.6397x; 1.4217x over previous
import functools

import jax
import jax.numpy as jnp
from jax import lax
from jax.experimental import pallas as pl
from jax.experimental.pallas import tpu as pltpu
from jax.experimental.pallas import tpu_sc as plsc

BATCH = 16384
EMBED_DIM = 64
H1 = 128
H2 = 64


@functools.cache
def _build_gather():
    info = plsc.get_sparse_core_info()
    nc, ns = info.num_cores, info.num_subcores
    nw = nc * ns
    bpw = BATCH // nw

    mesh = plsc.VectorSubcoreMesh(core_axis_name="c", subcore_axis_name="s")

    @functools.partial(
        pl.kernel,
        mesh=mesh,
        out_type=[
            jax.ShapeDtypeStruct((BATCH, EMBED_DIM), jnp.float32),
            jax.ShapeDtypeStruct((BATCH, EMBED_DIM), jnp.float32),
        ],
        scratch_types=[
            pltpu.VMEM((bpw,), jnp.int32),
            pltpu.VMEM((bpw,), jnp.int32),
            pltpu.VMEM((bpw, EMBED_DIM), jnp.float32),
            pltpu.SemaphoreType.DMA,
        ],
    )
    def gather2(u_tab, s_tab, u_idx, s_idx, u_out, s_out,
                u_idx_v, s_idx_v, rows_v, sem):
        wid = lax.axis_index("s") * nc + lax.axis_index("c")
        base = wid * bpw
        pltpu.sync_copy(u_idx.at[pl.ds(base, bpw)], u_idx_v)
        pltpu.sync_copy(s_idx.at[pl.ds(base, bpw)], s_idx_v)

        def u_body(b, _):
            k = b * 16
            v = u_idx_v[pl.ds(k, 16)]
            for j in range(16):
                pltpu.async_copy(
                    u_tab.at[pl.ds(v[j], 1)], rows_v.at[pl.ds(k + j, 1)], sem)
            return 0

        def s_body(b, _):
            k = b * 16
            v = s_idx_v[pl.ds(k, 16)]
            for j in range(16):
                pltpu.async_copy(
                    s_tab.at[pl.ds(v[j], 1)], rows_v.at[pl.ds(k + j, 1)], sem)
            return 0

        lax.fori_loop(0, bpw // 16, u_body, 0)
        pltpu.make_async_copy(u_tab.at[pl.ds(0, bpw)], rows_v, sem).wait()
        pltpu.sync_copy(rows_v, u_out.at[pl.ds(base, bpw)])

        lax.fori_loop(0, bpw // 16, s_body, 0)
        pltpu.make_async_copy(s_tab.at[pl.ds(0, bpw)], rows_v, sem).wait()
        pltpu.sync_copy(rows_v, s_out.at[pl.ds(base, bpw)])

    return gather2


def _mlp_body(ue_ref, se_ref, w1u_ref, w1s_ref, b1_ref, w2_ref, b2_ref,
              w3_ref, b3_ref, out_ref):
    h = jnp.dot(ue_ref[...], w1u_ref[...], preferred_element_type=jnp.float32)
    h += jnp.dot(se_ref[...], w1s_ref[...], preferred_element_type=jnp.float32)
    h = jnp.maximum(h + b1_ref[...], 0.0)
    h = jnp.dot(h, w2_ref[...], preferred_element_type=jnp.float32)
    h = jnp.maximum(h + b2_ref[...], 0.0)
    o = jnp.dot(h, w3_ref[...], preferred_element_type=jnp.float32)
    o = o + b3_ref[...]
    out_ref[...] = 1.0 / (1.0 + jnp.exp(-o))


def _mlp(u_emb, s_emb, w1uT, w1sT, b1, w2T, b2, w3T, b3):
    blk = 2048
    grid = BATCH // blk
    full = lambda shape: pl.BlockSpec(shape, lambda i: (0, 0))
    return pl.pallas_call(
        _mlp_body,
        grid=(grid,),
        in_specs=[
            pl.BlockSpec((blk, EMBED_DIM), lambda i: (i, 0)),
            pl.BlockSpec((blk, EMBED_DIM), lambda i: (i, 0)),
            full((EMBED_DIM, H1)),
            full((EMBED_DIM, H1)),
            full((1, H1)),
            full((H1, H2)),
            full((1, H2)),
            full((H2, 1)),
            full((1, 1)),
        ],
        out_specs=pl.BlockSpec((blk, 1), lambda i: (i, 0)),
        out_shape=jax.ShapeDtypeStruct((BATCH, 1), jnp.float32),
    )(u_emb, s_emb, w1uT, w1sT, b1, w2T, b2, w3T, b3)


def kernel(user, song, user_table, song_table, W1, b1, W2, b2, W3, b3):
    u_emb, s_emb = _build_gather()(
        user_table, song_table,
        user.astype(jnp.int32), song.astype(jnp.int32))
    w1uT = W1[:, :EMBED_DIM].T
    w1sT = W1[:, EMBED_DIM:].T
    return _mlp(u_emb, s_emb, w1uT, w1sT, b1.reshape(1, H1),
                W2.T, b2.reshape(1, H2), W3.T, b3.reshape(1, 1))


# speedup vs baseline: 1.7691x; 1.0789x over previous
import functools

import jax
import jax.numpy as jnp
from jax import lax
from jax.experimental import pallas as pl
from jax.experimental.pallas import tpu as pltpu
from jax.experimental.pallas import tpu_sc as plsc

BATCH = 16384
EMBED_DIM = 64
H1 = 128
H2 = 64


@functools.cache
def _build_gather():
    info = plsc.get_sparse_core_info()
    nc, ns = info.num_cores, info.num_subcores
    nw = nc * ns
    bpw = BATCH // nw

    mesh = plsc.VectorSubcoreMesh(core_axis_name="c", subcore_axis_name="s")

    @functools.partial(
        pl.kernel,
        mesh=mesh,
        out_type=[
            jax.ShapeDtypeStruct((BATCH, EMBED_DIM), jnp.float32),
            jax.ShapeDtypeStruct((BATCH, EMBED_DIM), jnp.float32),
        ],
        scratch_types=[
            pltpu.VMEM((bpw,), jnp.int32),
            pltpu.VMEM((bpw,), jnp.int32),
            pltpu.VMEM((bpw, EMBED_DIM), jnp.float32),
            pltpu.SemaphoreType.DMA,
        ],
    )
    def gather2(u_tab, s_tab, u_idx, s_idx, u_out, s_out,
                u_idx_v, s_idx_v, rows_v, sem):
        wid = lax.axis_index("s") * nc + lax.axis_index("c")
        base = wid * bpw
        pltpu.sync_copy(u_idx.at[pl.ds(base, bpw)], u_idx_v)
        pltpu.sync_copy(s_idx.at[pl.ds(base, bpw)], s_idx_v)

        def u_body(b, _):
            k = b * 16
            v = u_idx_v[pl.ds(k, 16)]
            for j in range(16):
                pltpu.async_copy(
                    u_tab.at[pl.ds(v[j], 1)], rows_v.at[pl.ds(k + j, 1)], sem)
            return 0

        def s_body(b, _):
            k = b * 16
            v = s_idx_v[pl.ds(k, 16)]
            for j in range(16):
                pltpu.async_copy(
                    s_tab.at[pl.ds(v[j], 1)], rows_v.at[pl.ds(k + j, 1)], sem)
            return 0

        lax.fori_loop(0, bpw // 16, u_body, 0)
        pltpu.make_async_copy(u_tab.at[pl.ds(0, bpw)], rows_v, sem).wait()
        pltpu.sync_copy(rows_v, u_out.at[pl.ds(base, bpw)])

        lax.fori_loop(0, bpw // 16, s_body, 0)
        pltpu.make_async_copy(s_tab.at[pl.ds(0, bpw)], rows_v, sem).wait()
        pltpu.sync_copy(rows_v, s_out.at[pl.ds(base, bpw)])

    return gather2


def _mlp_body(ue_ref, se_ref, w1u_ref, w1s_ref, b1_ref, w2_ref, b2_ref,
              w3_ref, b3_ref, out_ref):
    dn1 = (((1,), (1,)), ((), ()))
    dn0 = (((1,), (0,)), ((), ()))
    h = lax.dot_general(w1u_ref[...], ue_ref[...], dn1,
                        preferred_element_type=jnp.float32)
    h += lax.dot_general(w1s_ref[...], se_ref[...], dn1,
                         preferred_element_type=jnp.float32)
    h = jnp.maximum(h + b1_ref[...], 0.0)
    h = lax.dot_general(w2_ref[...], h, dn0,
                        preferred_element_type=jnp.float32)
    h = jnp.maximum(h + b2_ref[...], 0.0)
    o = lax.dot_general(w3_ref[...], h, dn0,
                        preferred_element_type=jnp.float32)
    o = o + b3_ref[...]
    out_ref[...] = 1.0 / (1.0 + jnp.exp(-o))


def _mlp(ue, se, W1u, W1s, b1, W2, b2, W3, b3):
    blk = 2048
    grid = BATCH // blk
    full = lambda shape: pl.BlockSpec(shape, lambda i: (0, 0))
    return pl.pallas_call(
        _mlp_body,
        grid=(grid,),
        in_specs=[
            pl.BlockSpec((blk, EMBED_DIM), lambda i: (i, 0)),
            pl.BlockSpec((blk, EMBED_DIM), lambda i: (i, 0)),
            full((H1, EMBED_DIM)),
            full((H1, EMBED_DIM)),
            full((H1, 1)),
            full((H2, H1)),
            full((H2, 1)),
            full((1, H2)),
            full((1, 1)),
        ],
        out_specs=pl.BlockSpec((1, blk), lambda i: (0, i)),
        out_shape=jax.ShapeDtypeStruct((1, BATCH), jnp.float32),
    )(ue, se, W1u, W1s, b1, W2, b2, W3, b3)


def kernel(user, song, user_table, song_table, W1, b1, W2, b2, W3, b3):
    ue, se = _build_gather()(user_table, song_table,
                             user.astype(jnp.int32), song.astype(jnp.int32))
    out = _mlp(ue, se, W1[:, :EMBED_DIM], W1[:, EMBED_DIM:], b1.reshape(H1, 1),
               W2, b2.reshape(H2, 1), W3, b3.reshape(1, 1))
    return out.reshape(BATCH, 1)
